# baseline (speedup 1.0000x reference)
import functools

import jax
import jax.numpy as jnp
from jax import lax
from jax.experimental import pallas as pl
from jax.experimental.pallas import tpu as pltpu
from jax.experimental.pallas import tpu_sc as plsc

_B, _D = 16384, 768
_L = 16
_NC, _NS = 2, 16
_NW = _NC * _NS
_BPW = _B // _NW
_CH = 8
_NCH = _BPW // _CH
_NB = 4
_NJ = _D // _L
_G = 4
_INV_D = 1.0 / _D

_GATHER_DNUMS = lax.GatherDimensionNumbers(
    offset_dims=(), collapsed_slice_dims=(0,), start_index_map=(0,))


def _lane_shuffle(v, idx):
    return lax.gather(v, idx[:, None], _GATHER_DNUMS, (1,),
                      mode=lax.GatherScatterMode.PROMISE_IN_BOUNDS)


def _lane_sum(v):
    lane = lax.iota(jnp.int32, _L)
    for sh in (8, 4, 2, 1):
        v = v + _lane_shuffle(v, lane ^ sh)
    return v


def _body(ids, table, gamma, beta, unit_flag, out, idx_v, rows_v, gamma_v,
          beta_v, flag_s, *sems):
    gsem = sems[:_NB]
    ssem = sems[_NB:]
    wid = lax.axis_index("s") * _NC + lax.axis_index("c")
    base = wid * _BPW
    pltpu.sync_copy(ids.at[pl.ds(base, _BPW)], idx_v)
    pltpu.sync_copy(gamma, gamma_v)
    pltpu.sync_copy(beta, beta_v)
    pltpu.sync_copy(unit_flag, flag_s)
    unit = flag_s[...][0] == 1

    def gather_copy(c, b):
        return pltpu.make_async_copy(
            table.at[idx_v.at[pl.ds(c * _CH, _CH)]],
            rows_v.at[pl.ds(b * _CH, _CH)], gsem[b])

    def store_copy(c, b):
        return pltpu.make_async_copy(
            rows_v.at[pl.ds(b * _CH, _CH)],
            out.at[pl.ds(base + c * _CH, _CH)], ssem[b])

    gather_copy(0, 0).start()
    gather_copy(1, 1).start()

    def chunk_body(c, carry):
        b = lax.rem(c, _NB)
        for bb in range(_NB):
            @pl.when(b == bb)
            def _(bb=bb):
                gather_copy(c, bb).wait()
                bf = (bb + 2) % _NB

                @pl.when(c >= 2)
                def _():
                    store_copy(c - 2, bf).wait()

                @pl.when(c + 2 < _NCH)
                def _():
                    gather_copy(c + 2, bf).start()

        def row_body(rg, affine=False):
            r0 = b * _CH + rg * _G
            s0 = [jnp.zeros((_L,), jnp.float32) for _ in range(_G)]
            s1 = [jnp.zeros((_L,), jnp.float32) for _ in range(_G)]
            q0 = [jnp.zeros((_L,), jnp.float32) for _ in range(_G)]
            q1 = [jnp.zeros((_L,), jnp.float32) for _ in range(_G)]
            for j in range(_NJ):
                sl = pl.ds(j * _L, _L)
                for g in range(_G):
                    v = rows_v[r0 + g, sl]
                    if j % 2 == 0:
                        s0[g] = s0[g] + v
                        q0[g] = q0[g] + v * v
                    else:
                        s1[g] = s1[g] + v
                        q1[g] = q1[g] + v * v
            m, y = [], []
            for g in range(_G):
                mg = _lane_sum(s0[g] + s1[g]) * _INV_D
                vv = (_lane_sum(q0[g] + q1[g]) * _INV_D
                      - mg * mg + 1e-5)
                iv = lax.bitcast_convert_type(vv, jnp.int32)
                yg = lax.bitcast_convert_type(
                    jnp.int32(0x5F3759DF) - (iv >> 1), jnp.float32)
                for _ in range(2):
                    yg = yg * (1.5 - 0.5 * vv * yg * yg)
                m.append(mg)
                y.append(yg)
            if affine:
                for j in range(_NJ):
                    sl = pl.ds(j * _L, _L)
                    gj = gamma_v[sl]
                    bj = beta_v[sl]
                    for g in range(_G):
                        v = rows_v[r0 + g, sl]
                        rows_v[r0 + g, sl] = (v - m[g]) * y[g] * gj + bj
            else:
                for j in range(_NJ):
                    sl = pl.ds(j * _L, _L)
                    for g in range(_G):
                        v = rows_v[r0 + g, sl]
                        rows_v[r0 + g, sl] = (v - m[g]) * y[g]

        @pl.when(unit)
        def _():
            plsc.parallel_loop(0, _CH // _G)(row_body)

        @pl.when(jnp.logical_not(unit))
        def _():
            plsc.parallel_loop(0, _CH // _G)(
                functools.partial(row_body, affine=True))

        for bb in range(_NB):
            @pl.when(b == bb)
            def _(bb=bb):
                store_copy(c, bb).start()
        return carry

    lax.fori_loop(0, _NCH, chunk_body, 0)
    store_copy(_NCH - 2, (_NCH - 2) % _NB).wait()
    store_copy(_NCH - 1, (_NCH - 1) % _NB).wait()


_encode = functools.partial(
    pl.kernel,
    out_type=jax.ShapeDtypeStruct((_B, _D), jnp.float32),
    mesh=plsc.VectorSubcoreMesh(core_axis_name="c", subcore_axis_name="s",
                                num_cores=_NC, num_subcores=_NS),
    scratch_types=[
        pltpu.VMEM((_BPW,), jnp.int32),
        pltpu.VMEM((_NB * _CH, _D), jnp.float32),
        pltpu.VMEM((_D,), jnp.float32),
        pltpu.VMEM((_D,), jnp.float32),
        pltpu.VMEM((_L,), jnp.int32),
    ] + [pltpu.SemaphoreType.DMA] * (2 * _NB),
)(_body)


def kernel(user_ids, table, gamma, beta):
    unit_flag = jnp.full((_L,), jnp.logical_and(
        jnp.all(gamma == 1.0), jnp.all(beta == 0.0)).astype(jnp.int32))
    return _encode(user_ids.astype(jnp.int32), table, gamma, beta, unit_flag)


# speedup vs baseline: 1.0492x; 1.0492x over previous
import functools

import jax
import jax.numpy as jnp
from jax import lax
from jax.experimental import pallas as pl
from jax.experimental.pallas import tpu as pltpu
from jax.experimental.pallas import tpu_sc as plsc

_B, _D = 16384, 768
_L = 16
_NC, _NS = 2, 16
_NW = _NC * _NS
_BPW = _B // _NW
_CH = 16
_NCH = _BPW // _CH
_NB = 4
_NJ = _D // _L
_G = 4
_INV_D = 1.0 / _D

_GATHER_DNUMS = lax.GatherDimensionNumbers(
    offset_dims=(), collapsed_slice_dims=(0,), start_index_map=(0,))


def _lane_shuffle(v, idx):
    return lax.gather(v, idx[:, None], _GATHER_DNUMS, (1,),
                      mode=lax.GatherScatterMode.PROMISE_IN_BOUNDS)


def _lane_sum(v):
    lane = lax.iota(jnp.int32, _L)
    for sh in (8, 4, 2, 1):
        v = v + _lane_shuffle(v, lane ^ sh)
    return v


def _body(ids, table, gamma, beta, unit_flag, out, idx_v, rows_v, gamma_v,
          beta_v, flag_s, *sems):
    gsem = sems[:_NB]
    ssem = sems[_NB:]
    wid = lax.axis_index("s") * _NC + lax.axis_index("c")
    base = wid * _BPW
    pltpu.sync_copy(ids.at[pl.ds(base, _BPW)], idx_v)
    pltpu.sync_copy(gamma, gamma_v)
    pltpu.sync_copy(beta, beta_v)
    pltpu.sync_copy(unit_flag, flag_s)
    unit = flag_s[...][0] == 1

    def gather_copy(c, b):
        return pltpu.make_async_copy(
            table.at[idx_v.at[pl.ds(c * _CH, _CH)]],
            rows_v.at[pl.ds(b * _CH, _CH)], gsem[b])

    def store_copy(c, b):
        return pltpu.make_async_copy(
            rows_v.at[pl.ds(b * _CH, _CH)],
            out.at[pl.ds(base + c * _CH, _CH)], ssem[b])

    gather_copy(0, 0).start()
    gather_copy(1, 1).start()

    def chunk_body(c, carry):
        b = lax.rem(c, _NB)
        for bb in range(_NB):
            @pl.when(b == bb)
            def _(bb=bb):
                gather_copy(c, bb).wait()
                bf = (bb + 2) % _NB

                @pl.when(c >= 2)
                def _():
                    store_copy(c - 2, bf).wait()

                @pl.when(c + 2 < _NCH)
                def _():
                    gather_copy(c + 2, bf).start()

        def row_body(rg, affine=False):
            r0 = b * _CH + rg * _G
            s0 = [jnp.zeros((_L,), jnp.float32) for _ in range(_G)]
            s1 = [jnp.zeros((_L,), jnp.float32) for _ in range(_G)]
            q0 = [jnp.zeros((_L,), jnp.float32) for _ in range(_G)]
            q1 = [jnp.zeros((_L,), jnp.float32) for _ in range(_G)]
            for j in range(_NJ):
                sl = pl.ds(j * _L, _L)
                for g in range(_G):
                    v = rows_v[r0 + g, sl]
                    if j % 2 == 0:
                        s0[g] = s0[g] + v
                        q0[g] = q0[g] + v * v
                    else:
                        s1[g] = s1[g] + v
                        q1[g] = q1[g] + v * v
            m, y = [], []
            for g in range(_G):
                mg = _lane_sum(s0[g] + s1[g]) * _INV_D
                vv = (_lane_sum(q0[g] + q1[g]) * _INV_D
                      - mg * mg + 1e-5)
                iv = lax.bitcast_convert_type(vv, jnp.int32)
                yg = lax.bitcast_convert_type(
                    jnp.int32(0x5F3759DF) - (iv >> 1), jnp.float32)
                for _ in range(2):
                    yg = yg * (1.5 - 0.5 * vv * yg * yg)
                m.append(mg)
                y.append(yg)
            if affine:
                for j in range(_NJ):
                    sl = pl.ds(j * _L, _L)
                    gj = gamma_v[sl]
                    bj = beta_v[sl]
                    for g in range(_G):
                        v = rows_v[r0 + g, sl]
                        rows_v[r0 + g, sl] = (v - m[g]) * y[g] * gj + bj
            else:
                for j in range(_NJ):
                    sl = pl.ds(j * _L, _L)
                    for g in range(_G):
                        v = rows_v[r0 + g, sl]
                        rows_v[r0 + g, sl] = (v - m[g]) * y[g]

        @pl.when(unit)
        def _():
            plsc.parallel_loop(0, _CH // _G)(row_body)

        @pl.when(jnp.logical_not(unit))
        def _():
            plsc.parallel_loop(0, _CH // _G)(
                functools.partial(row_body, affine=True))

        for bb in range(_NB):
            @pl.when(b == bb)
            def _(bb=bb):
                store_copy(c, bb).start()
        return carry

    lax.fori_loop(0, _NCH, chunk_body, 0)
    store_copy(_NCH - 2, (_NCH - 2) % _NB).wait()
    store_copy(_NCH - 1, (_NCH - 1) % _NB).wait()


_encode = functools.partial(
    pl.kernel,
    out_type=jax.ShapeDtypeStruct((_B, _D), jnp.float32),
    mesh=plsc.VectorSubcoreMesh(core_axis_name="c", subcore_axis_name="s",
                                num_cores=_NC, num_subcores=_NS),
    scratch_types=[
        pltpu.VMEM((_BPW,), jnp.int32),
        pltpu.VMEM((_NB * _CH, _D), jnp.float32),
        pltpu.VMEM((_D,), jnp.float32),
        pltpu.VMEM((_D,), jnp.float32),
        pltpu.VMEM((_L,), jnp.int32),
    ] + [pltpu.SemaphoreType.DMA] * (2 * _NB),
)(_body)


def kernel(user_ids, table, gamma, beta):
    unit_flag = jnp.full((_L,), jnp.logical_and(
        jnp.all(gamma == 1.0), jnp.all(beta == 0.0)).astype(jnp.int32))
    return _encode(user_ids.astype(jnp.int32), table, gamma, beta, unit_flag)
